# baseline (speedup 1.0000x reference)
import jax
import jax.numpy as jnp
from jax.experimental import pallas as pl
from jax.experimental.pallas import tpu as pltpu

_B = 2
_N = 64 * 64
_ROWS = 1024
_NBLK = _B * _N // _ROWS


def _build_adjacency(out_ref, zeros_ref, sem):
    zeros_ref[...] = jnp.zeros_like(zeros_ref)
    copies = [
        pltpu.make_async_copy(
            zeros_ref, out_ref.at[pl.ds(i * _ROWS, _ROWS), :], sem.at[i]
        )
        for i in range(_NBLK)
    ]
    for cp in copies:
        cp.start()
    for cp in copies:
        cp.wait()


def kernel(data):
    flat = pl.pallas_call(
        _build_adjacency,
        out_specs=pl.BlockSpec(memory_space=pl.ANY),
        out_shape=jax.ShapeDtypeStruct((_B * _N, _N), jnp.float32),
        scratch_shapes=[
            pltpu.VMEM((_ROWS, _N), jnp.float32),
            pltpu.SemaphoreType.DMA((_NBLK,)),
        ],
    )()
    return (flat.reshape(_B, _N, _N), data)


# speedup vs baseline: 1.0135x; 1.0135x over previous
import jax
import jax.numpy as jnp
from jax.experimental import pallas as pl
from jax.experimental.pallas import tpu as pltpu

_B = 2
_N = 64 * 64
_ROWS = 512
_NBLK = _B * _N // _ROWS


def _build_adjacency(out_ref, zeros_ref, sem):
    zeros_ref[...] = jnp.zeros_like(zeros_ref)
    copies = [
        pltpu.make_async_copy(
            zeros_ref, out_ref.at[pl.ds(i * _ROWS, _ROWS), :], sem.at[i]
        )
        for i in range(_NBLK)
    ]
    for cp in copies:
        cp.start()
    for cp in copies:
        cp.wait()


def kernel(data):
    flat = pl.pallas_call(
        _build_adjacency,
        out_specs=pl.BlockSpec(memory_space=pl.ANY),
        out_shape=jax.ShapeDtypeStruct((_B * _N, _N), jnp.float32),
        scratch_shapes=[
            pltpu.VMEM((_ROWS, _N), jnp.float32),
            pltpu.SemaphoreType.DMA((_NBLK,)),
        ],
    )()
    return (flat.reshape(_B, _N, _N), data)
